# Initial kernel scaffold; baseline (speedup 1.0000x reference)
import jax
import jax.numpy as jnp
from jax.experimental import pallas as pl


def kernel(node_x, net_x, edge_index_sink_to_net, edge_index_source_to_net, edge_weight, edge_attr, W_ne1, b_ne1, W_ne2, b_ne2, W_te1, b_te1, W_te2, b_te2, W_net, b_net, W_node, b_node, W_attr, b_attr, W_f1n, b_f1n, W_f2n, b_f2n, W_f1e, b_f1e, W_f2e, b_f2e):
    raise NotImplementedError("write your pallas kernel here")


# speedup vs baseline: 2.1585x; 2.1585x over previous
import functools

import jax
import jax.numpy as jnp
from jax import lax
from jax.experimental import pallas as pl
from jax.experimental.pallas import tpu as pltpu
from jax.experimental.pallas import tpu_sc as plsc

NC = 2
NS = 16
NW = NC * NS
LANES = 16
D = 64
ACC_ROWS = 10240


def _leaky(x):
    return jnp.where(x >= 0, x, 0.01 * x)


G = 16


def _seg_sum_sc(tab, gi, si, w, n_chunks, two_part):
    d_tab = tab.shape[1]
    mesh = plsc.VectorSubcoreMesh(core_axis_name="c", subcore_axis_name="s")

    @functools.partial(
        pl.kernel,
        out_type=jax.ShapeDtypeStruct((NC, ACC_ROWS, D), jnp.float32),
        mesh=mesh,
        compiler_params=pltpu.CompilerParams(needs_layout_passes=False,
                                             use_tc_tiling_on_sc=False),
        scratch_types=[
            pltpu.VMEM((G, 128), jnp.int32),
            pltpu.VMEM((G, 128), jnp.int32),
            pltpu.VMEM((G, 128), jnp.float32),
            pltpu.VMEM((128, d_tab), jnp.float32),
            pltpu.VMEM((128, D), jnp.float32),
            pltpu.VMEM_SHARED((ACC_ROWS, D), jnp.float32),
        ],
    )
    def k(tab_hbm, gi_hbm, si_hbm, w_hbm, out_hbm, gi_v, si_v, w_v, rows_v,
          contrib_v, acc_s):
        c = lax.axis_index("c")
        s = lax.axis_index("s")
        wid = c * NS + s

        zero = jnp.zeros((LANES,), jnp.float32)

        @pl.loop(0, 128)
        def _(i):
            for j in range(D // LANES):
                contrib_v[i, pl.ds(j * LANES, LANES)] = zero

        rpt = ACC_ROWS // NS

        @pl.loop(0, rpt, step=128)
        def _(r):
            pltpu.sync_copy(contrib_v, acc_s.at[pl.ds(s * rpt + r, 128)])

        plsc.subcore_barrier()

        @pl.loop(0, n_chunks, step=G)
        def _(g):
            pltpu.sync_copy(gi_hbm.at[wid].at[pl.ds(g, G)], gi_v)
            pltpu.sync_copy(si_hbm.at[wid].at[pl.ds(g, G)], si_v)
            pltpu.sync_copy(w_hbm.at[wid].at[pl.ds(g, G)], w_v)

            @pl.loop(0, G)
            def _(cc):
                pltpu.sync_copy(tab_hbm.at[gi_v.at[cc]], rows_v)

                @pl.loop(0, 128)
                def _(e):
                    wv = plsc.load_gather(
                        w_v,
                        [jnp.full((LANES,), cc, jnp.int32),
                         jnp.full((LANES,), e, jnp.int32)],
                    )
                    for j in range(D // LANES):
                        a = rows_v[e, pl.ds(j * LANES, LANES)]
                        if two_part:
                            b = rows_v[e, pl.ds(D + j * LANES, LANES)]
                            contrib_v[e, pl.ds(j * LANES, LANES)] = a * wv + b
                        else:
                            contrib_v[e, pl.ds(j * LANES, LANES)] = a * wv

                pltpu.sync_copy(contrib_v, acc_s.at[si_v.at[cc]], add=True)

        plsc.subcore_barrier()

        pltpu.sync_copy(acc_s.at[pl.ds(s * rpt, rpt)],
                        out_hbm.at[c].at[pl.ds(s * rpt, rpt)])

    return k(tab, gi, si, w)


def _mlp2(x, W1, b1, W2, b2, block, abs_out):
    n, din = x.shape
    h = W1.shape[1]
    dout = W2.shape[1]

    def body(x_ref, w1_ref, b1_ref, w2_ref, b2_ref, o_ref):
        hh = jnp.dot(x_ref[...], w1_ref[...],
                     preferred_element_type=jnp.float32) + b1_ref[...]
        hh = _leaky(hh)
        z = jnp.dot(hh, w2_ref[...],
                    preferred_element_type=jnp.float32) + b2_ref[...]
        o_ref[...] = jnp.abs(z) if abs_out else z

    return pl.pallas_call(
        body,
        grid=(n // block,),
        in_specs=[
            pl.BlockSpec((block, din), lambda i: (i, 0)),
            pl.BlockSpec((din, h), lambda i: (0, 0)),
            pl.BlockSpec((1, h), lambda i: (0, 0)),
            pl.BlockSpec((h, dout), lambda i: (0, 0)),
            pl.BlockSpec((1, dout), lambda i: (0, 0)),
        ],
        out_specs=pl.BlockSpec((block, dout), lambda i: (i, 0)),
        out_shape=jax.ShapeDtypeStruct((n, dout), jnp.float32),
    )(x, W1.reshape(din, h), b1.reshape(1, h), W2.reshape(h, dout),
      b2.reshape(1, dout))


def _net_update(h_net, acc, W_top, W_bot, b, wa, ba, block):
    n = h_net.shape[0]

    def body(h_ref, a0_ref, a1_ref, wt_ref, wb_ref, b_ref, wa_ref, ba_ref,
             hn_ref, t_ref):
        hh = h_ref[...]
        agg = a0_ref[0] + a1_ref[0]
        z = (jnp.dot(hh, wt_ref[...], preferred_element_type=jnp.float32)
             + jnp.dot(agg, wb_ref[...], preferred_element_type=jnp.float32)
             + b_ref[...])
        hnew = _leaky(z) + hh
        hn_ref[...] = _leaky(hnew)
        t_ref[...] = jnp.concatenate(
            [hnew * wa_ref[...], hnew * ba_ref[...]], axis=1)

    return pl.pallas_call(
        body,
        grid=(n // block,),
        in_specs=[
            pl.BlockSpec((block, D), lambda i: (i, 0)),
            pl.BlockSpec((1, block, D), lambda i: (0, i, 0)),
            pl.BlockSpec((1, block, D), lambda i: (1, i, 0)),
            pl.BlockSpec((D, D), lambda i: (0, 0)),
            pl.BlockSpec((D, D), lambda i: (0, 0)),
            pl.BlockSpec((1, D), lambda i: (0, 0)),
            pl.BlockSpec((1, D), lambda i: (0, 0)),
            pl.BlockSpec((1, D), lambda i: (0, 0)),
        ],
        out_specs=[
            pl.BlockSpec((block, D), lambda i: (i, 0)),
            pl.BlockSpec((block, 2 * D), lambda i: (i, 0)),
        ],
        out_shape=[
            jax.ShapeDtypeStruct((n, D), jnp.float32),
            jax.ShapeDtypeStruct((n, 2 * D), jnp.float32),
        ],
    )(h_net, acc, acc, W_top, W_bot, b.reshape(1, D), wa.reshape(1, D),
      ba.reshape(1, D))


def _node_update(h_inst, acc, W_top, W_bot, b, block, n_agg_rows):
    n = h_inst.shape[0]
    n_agg_blocks = n_agg_rows // block

    def body(h_ref, a0_ref, a1_ref, wt_ref, wb_ref, b_ref, o_ref):
        i = pl.program_id(0)
        hh = h_ref[...]
        agg = a0_ref[0] + a1_ref[0]
        gate = jnp.where(i < n_agg_blocks, 1.0, 0.0).astype(jnp.float32)
        z = (jnp.dot(hh, wt_ref[...], preferred_element_type=jnp.float32)
             + jnp.dot(agg, wb_ref[...],
                       preferred_element_type=jnp.float32) * gate
             + b_ref[...])
        hnew = _leaky(z) + hh
        o_ref[...] = _leaky(hnew)

    clamp = n_agg_blocks - 1

    return pl.pallas_call(
        body,
        grid=(n // block,),
        in_specs=[
            pl.BlockSpec((block, D), lambda i: (i, 0)),
            pl.BlockSpec((1, block, D), lambda i: (0, jnp.minimum(i, clamp), 0)),
            pl.BlockSpec((1, block, D), lambda i: (1, jnp.minimum(i, clamp), 0)),
            pl.BlockSpec((D, D), lambda i: (0, 0)),
            pl.BlockSpec((D, D), lambda i: (0, 0)),
            pl.BlockSpec((1, D), lambda i: (0, 0)),
        ],
        out_specs=pl.BlockSpec((block, D), lambda i: (i, 0)),
        out_shape=jax.ShapeDtypeStruct((n, D), jnp.float32),
    )(h_inst, acc, acc, W_top, W_bot, b.reshape(1, D))


def _pack_edges(g, s, w, pad_seg):
    e = g.shape[0]
    k = -(-e // (NW * 128 * G)) * G
    tot = NW * 128 * k
    pad = tot - e
    g = jnp.concatenate([g, jnp.zeros((pad,), jnp.int32)])
    s = jnp.concatenate([s, jnp.full((pad,), pad_seg, jnp.int32)])
    w = jnp.concatenate([w, jnp.zeros((pad,), jnp.float32)])
    return (g.reshape(NW, k, 128), s.reshape(NW, k, 128),
            w.reshape(NW, k, 128), k)


def kernel(node_x, net_x, edge_index_sink_to_net, edge_index_source_to_net,
           edge_weight, edge_attr, W_ne1, b_ne1, W_ne2, b_ne2, W_te1, b_te1,
           W_te2, b_te2, W_net, b_net, W_node, b_node, W_attr, b_attr, W_f1n,
           b_f1n, W_f2n, b_f2n, W_f1e, b_f1e, W_f2e, b_f2e):
    n_node = node_x.shape[0]
    n_net = net_x.shape[0]
    n_layers = W_net.shape[0]
    e2 = edge_index_source_to_net.shape[1]

    sink_node = edge_index_sink_to_net[0].astype(jnp.int32)
    sink_net = edge_index_sink_to_net[1].astype(jnp.int32)
    src_node = edge_index_source_to_net[0].astype(jnp.int32)
    src_net = edge_index_source_to_net[1].astype(jnp.int32)

    gn, sn, wn, k1 = _pack_edges(
        jnp.concatenate([sink_node, src_node]),
        jnp.concatenate([sink_net, src_net]),
        jnp.concatenate([edge_weight, jnp.ones((e2,), jnp.float32)]),
        n_net)

    gv, sv, wv, k2 = _pack_edges(sink_net, sink_node, edge_attr[:, 0], n_net)

    h_inst = _mlp2(node_x, W_ne1, b_ne1, W_ne2, b_ne2, block=2000,
                   abs_out=False)
    h_net = _mlp2(net_x, W_te1, b_te1, W_te2, b_te2, block=2000,
                  abs_out=False)

    for l in range(n_layers):
        acc_net = _seg_sum_sc(h_inst[:n_net], gn, sn, wn, k1, two_part=False)
        h_net, tab = _net_update(h_net, acc_net, W_net[l][:D], W_net[l][D:],
                                 b_net[l], W_attr[l, 0], b_attr[l], block=2000)
        acc_node = _seg_sum_sc(tab, gv, sv, wv, k2, two_part=True)
        h_inst = _node_update(h_inst, acc_node, W_node[l][:D], W_node[l][D:],
                              b_node[l], block=2000, n_agg_rows=n_net)

    node_rep = _mlp2(h_inst, W_f1n, b_f1n, W_f2n, b_f2n, block=2000,
                     abs_out=True)
    net_rep = _mlp2(h_net, W_f1e, b_f1e, W_f2e, b_f2e, block=2000,
                    abs_out=True)
    return node_rep, net_rep
